# baseline (speedup 1.0000x reference)
import functools

import jax
import jax.numpy as jnp
from jax import lax
from jax.experimental import pallas as pl
from jax.experimental.pallas import tpu as pltpu
from jax.experimental.pallas import tpu_sc as plsc

D_MODEL_ = 4096
NUM_EXPERTS_ = 64
CAPACITY_FACTOR_ = 1.0
EPS_ = 1e-06
NUM_TOKENS_ = 8192
BT_ = 1024

NC_ = 2
NS_ = 16
NW_ = NC_ * NS_
ROWS_ = NUM_TOKENS_ // NW_


def _tc_kernel(x_ref, w_ref, out_ref, denom_ref):
    i = pl.program_id(0)

    gt = jax.lax.dot_general(
        w_ref[...], x_ref[...],
        dimension_numbers=(((1,), (1,)), ((), ())),
        preferred_element_type=jnp.float32,
    )

    mx = jnp.max(gt, axis=0, keepdims=True)
    rows = jax.lax.broadcasted_iota(jnp.int32, gt.shape, 0)
    eq = gt == mx
    first = jnp.min(jnp.where(eq, rows, NUM_EXPERTS_), axis=0, keepdims=True)
    masked_t = jnp.where(rows == first, gt, 0.0)

    out_ref[...] = masked_t.T
    part = jnp.sum(masked_t, axis=1, keepdims=True).T

    @pl.when(i == 0)
    def _init():
        denom_ref[...] = part

    @pl.when(i != 0)
    def _accum():
        denom_ref[...] += part


def _tc_pass(x, W):
    n_tokens = x.shape[0]
    return pl.pallas_call(
        _tc_kernel,
        grid=(n_tokens // BT_,),
        in_specs=[
            pl.BlockSpec((BT_, D_MODEL_), lambda i: (i, 0)),
            pl.BlockSpec((NUM_EXPERTS_, D_MODEL_), lambda i: (0, 0)),
        ],
        out_specs=[
            pl.BlockSpec((BT_, NUM_EXPERTS_), lambda i: (i, 0)),
            pl.BlockSpec((1, NUM_EXPERTS_), lambda i: (0, 0)),
        ],
        out_shape=[
            jax.ShapeDtypeStruct((n_tokens, NUM_EXPERTS_), jnp.float32),
            jax.ShapeDtypeStruct((1, NUM_EXPERTS_), jnp.float32),
        ],
    )(x, W)


def _sc_scale_kernel(masked_hbm, denom_hbm, out_hbm, rows_v, dvec_v, sem):
    wid = lax.axis_index("s") * NC_ + lax.axis_index("c")
    base = wid * ROWS_
    pltpu.sync_copy(denom_hbm.at[0], dvec_v)
    copy = pltpu.make_async_copy(
        masked_hbm.at[pl.ds(base, ROWS_), :], rows_v, sem)
    copy.start()

    capacity = jnp.float32(float(CAPACITY_FACTOR_ * NUM_TOKENS_))
    svs = []
    for c in range(NUM_EXPERTS_ // 16):
        d = dvec_v[pl.ds(c * 16, 16)]
        svs.append(capacity / (d + EPS_))

    copy.wait()

    def body(r, carry):
        for c in range(NUM_EXPERTS_ // 16):
            sl = pl.ds(c * 16, 16)
            rows_v[r, sl] = rows_v[r, sl] * svs[c]
        return carry

    lax.fori_loop(0, ROWS_, body, jnp.int32(0))

    pltpu.sync_copy(rows_v, out_hbm.at[pl.ds(base, ROWS_), :])


@functools.partial(
    pl.kernel,
    out_type=jax.ShapeDtypeStruct((NUM_TOKENS_, NUM_EXPERTS_), jnp.float32),
    mesh=plsc.VectorSubcoreMesh(core_axis_name="c", subcore_axis_name="s"),
    scratch_types=[
        pltpu.VMEM((ROWS_, NUM_EXPERTS_), jnp.float32),
        pltpu.VMEM((NUM_EXPERTS_,), jnp.float32),
        pltpu.SemaphoreType.DMA,
    ],
)
def _sc_pass(masked_hbm, denom_hbm, out_hbm, rows_v, dvec_v, sem):
    _sc_scale_kernel(masked_hbm, denom_hbm, out_hbm, rows_v, dvec_v, sem)


@functools.partial(jax.jit)
def kernel(x, W):
    masked, denom = _tc_pass(x, W)
    return _sc_pass(masked, denom)


# speedup vs baseline: 1.4516x; 1.4516x over previous
import functools

import jax
import jax.numpy as jnp
from jax.experimental import pallas as pl
from jax.experimental.pallas import tpu as pltpu

D_MODEL_ = 4096
NUM_EXPERTS_ = 64
CAPACITY_FACTOR_ = 1.0
EPS_ = 1e-06
NUM_TOKENS_ = 8192
BT_ = 512


def _route_block(gt):
    mx = jnp.max(gt, axis=0, keepdims=True)
    rows = jax.lax.broadcasted_iota(jnp.int32, gt.shape, 0)
    eq = gt == mx
    first = jnp.min(jnp.where(eq, rows, NUM_EXPERTS_), axis=0, keepdims=True)
    return jnp.where(rows == first, gt, 0.0)


def _router_kernel(x0_ref, x1_ref, w_ref, out_ref, denom_ref):
    i = pl.program_id(0)
    nsteps = pl.num_programs(0)
    half = nsteps * BT_

    w = w_ref[...]
    gt0 = jax.lax.dot_general(
        w, x0_ref[...], dimension_numbers=(((1,), (1,)), ((), ())),
        preferred_element_type=jnp.float32)
    gt1 = jax.lax.dot_general(
        w, x1_ref[...], dimension_numbers=(((1,), (1,)), ((), ())),
        preferred_element_type=jnp.float32)

    m0 = _route_block(gt0)
    m1 = _route_block(gt1)

    out_ref[pl.ds(i * BT_, BT_), :] = m0.T
    out_ref[pl.ds(half + i * BT_, BT_), :] = m1.T

    part = (jnp.sum(m0, axis=1, keepdims=True)
            + jnp.sum(m1, axis=1, keepdims=True))

    @pl.when(i == 0)
    def _init():
        denom_ref[...] = part

    @pl.when(i != 0)
    def _accum():
        denom_ref[...] += part

    @pl.when(i == nsteps - 1)
    def _finalize():
        capacity = jnp.float32(int(CAPACITY_FACTOR_ * NUM_TOKENS_))
        scale = capacity / (denom_ref[...] + EPS_)
        out_ref[...] = out_ref[...] * scale.T


@functools.partial(jax.jit)
def kernel(x, W):
    n_tokens = x.shape[0]
    nsteps = n_tokens // (2 * BT_)
    return pl.pallas_call(
        _router_kernel,
        grid=(nsteps,),
        in_specs=[
            pl.BlockSpec((BT_, D_MODEL_), lambda i: (i, 0)),
            pl.BlockSpec((BT_, D_MODEL_), lambda i, _n=nsteps: (i + _n, 0)),
            pl.BlockSpec((NUM_EXPERTS_, D_MODEL_), lambda i: (0, 0)),
        ],
        out_specs=pl.BlockSpec((n_tokens, NUM_EXPERTS_), lambda i: (0, 0)),
        out_shape=jax.ShapeDtypeStruct((n_tokens, NUM_EXPERTS_), jnp.float32),
        scratch_shapes=[pltpu.VMEM((NUM_EXPERTS_, 1), jnp.float32)],
    )(x, x, W)
